# baseline (speedup 1.0000x reference)
import functools

import jax
import jax.numpy as jnp
from jax import lax
from jax.experimental import pallas as pl
from jax.experimental.pallas import tpu as pltpu
from jax.experimental.pallas import tpu_sc as plsc

N_NODES = 10000
N_EDGES = 320000
IN_FEATS = 128
N_HIDDEN = 128
N_CLASSES = 64

_NC = 2
_NS = 16
_NW = _NC * _NS
_CH = 64
_NCH = 160
_NQ = _NCH // 4
_EPT = _NCH * _CH
_EPAD = _NW * _EPT
_NPAD = 10240
_RPT = _NPAD // _NS
_DH = 64


def _make_agg(with_deg, n_halves=2):
    mesh = plsc.VectorSubcoreMesh(core_axis_name="c", subcore_axis_name="s")

    out_type = [jax.ShapeDtypeStruct((_NC, _NPAD, _DH), jnp.float32)
                for _ in range(n_halves)]
    scratch = [
        pltpu.VMEM((_EPT,), jnp.int32),
        pltpu.VMEM((_EPT,), jnp.int32),
    ] + [pltpu.VMEM((_CH,), jnp.int32) for _ in range(4)] + [
        pltpu.VMEM((_CH, _DH), jnp.float32) for _ in range(4)
    ] + [
        pltpu.VMEM_SHARED((_NPAD, _DH), jnp.float32),
        pltpu.VMEM_SHARED((_NPAD, _DH), jnp.float32),
    ] + [pltpu.SemaphoreType.DMA for _ in range(8)]
    if with_deg:
        out_type.append(jax.ShapeDtypeStruct((_NW * _NPAD,), jnp.float32))
        scratch.append(pltpu.VMEM((_NPAD,), jnp.float32))

    @functools.partial(
        pl.kernel,
        out_type=out_type,
        mesh=mesh,
        scratch_types=scratch,
        compiler_params=pltpu.CompilerParams(
            needs_layout_passes=False, use_tc_tiling_on_sc=False),
    )
    def agg(*args):
        t_hbms = args[:n_halves]
        src_hbm, dst_hbm, zeros_hbm = args[n_halves:n_halves + 3]
        refs = args[n_halves + 3:]
        o_hbms = refs[:n_halves]
        refs = refs[n_halves:]
        if with_deg:
            deg_hbm = refs[0]
            refs = refs[1:]
        srcf, dstf = refs[0], refs[1]
        dstbufs = refs[2:6]
        rows = refs[6:10]
        table, accum = refs[10], refs[11]
        gsems = refs[12:16]
        ssems = refs[16:20]
        if with_deg:
            degv = refs[20]
        c = lax.axis_index("c")
        s = lax.axis_index("s")
        wid = s * _NC + c
        myrows = pl.ds(s * _RPT, _RPT)
        ones16 = jnp.ones((16,), jnp.float32)

        def gather(i, buf, sem):
            idx = srcf.at[pl.ds(pl.multiple_of(i * _CH, _CH), _CH)]
            pltpu.async_copy(table.at[idx], buf, sem)

        def gwait(i, buf, sem):
            idx = srcf.at[pl.ds(pl.multiple_of(i * _CH, _CH), _CH)]
            pltpu.make_async_copy(table.at[idx], buf, sem).wait()

        for half in range(n_halves):
            t_hbm = t_hbms[half]
            out_hbm = o_hbms[half]
            pltpu.sync_copy(t_hbm.at[myrows], table.at[myrows])
            pltpu.sync_copy(zeros_hbm, accum.at[myrows])
            if half == 0:
                pltpu.sync_copy(src_hbm.at[pl.ds(wid * _EPT, _EPT)], srcf)
                pltpu.sync_copy(dst_hbm.at[pl.ds(wid * _EPT, _EPT)], dstf)
                if with_deg:
                    def zstep(i, carry):
                        degv[pl.ds(i * 16, 16)] = jnp.zeros((16,), jnp.float32)
                        return carry
                    lax.fori_loop(0, _NPAD // 16, zstep, 0)
            plsc.subcore_barrier()

            track_deg = with_deg and half == 0

            def fill_dstbuf(i, db):
                for j in range(_CH // 16):
                    v = dstf[pl.ds(i * _CH + j * 16, 16)]
                    db[pl.ds(j * 16, 16)] = v
                    if track_deg:
                        plsc.addupdate_scatter(degv, [v], ones16)

            for b in range(4):
                gather(b, rows[b], gsems[b])

            def quad(q, carry):
                c0 = 4 * q
                for b in range(4):
                    gwait(c0 + b, rows[b], gsems[b])
                    fill_dstbuf(c0 + b, dstbufs[b])
                    pltpu.async_copy(rows[b], accum.at[dstbufs[b]],
                                     ssems[b], add=True)
                for b in range(4):
                    pltpu.make_async_copy(
                        rows[b], accum.at[dstbufs[b]], ssems[b]).wait()

                    @pl.when(q < _NQ - 1)
                    def _(b=b):
                        gather(c0 + 4 + b, rows[b], gsems[b])
                return carry

            lax.fori_loop(0, _NQ, quad, 0)
            plsc.subcore_barrier()
            pltpu.sync_copy(accum.at[myrows], out_hbm.at[c, myrows])

        if with_deg:
            pltpu.sync_copy(degv, deg_hbm.at[pl.ds(wid * _NPAD, _NPAD)])

    return agg


_agg_deg = _make_agg(True)
_agg = _make_agg(False)
_agg1 = _make_agg(False, n_halves=1)

_BLK = 1024
_GRID = (_NPAD // _BLK,)


def _deg_body(parts_ref, deg_ref):
    deg_ref[...] = jnp.sum(parts_ref[...], axis=0, keepdims=True)


_tc_deg = pl.pallas_call(
    _deg_body,
    grid=_GRID,
    in_specs=[pl.BlockSpec((_NW, _BLK), lambda i: (0, i))],
    out_specs=pl.BlockSpec((1, _BLK), lambda i: (0, i)),
    out_shape=jax.ShapeDtypeStruct((1, _NPAD), jnp.float32),
)


def _hidden_sage(x_ref, plo_ref, phi_ref, deg_ref, ws_ref, wnlo_ref,
                 wnhi_ref, b_ref):
    rdeg = 1.0 / jnp.maximum(deg_ref[...], 1.0)
    agg_lo = (plo_ref[0] + plo_ref[1]) * rdeg
    agg_hi = (phi_ref[0] + phi_ref[1]) * rdeg
    h = (jnp.dot(x_ref[...], ws_ref[...], preferred_element_type=jnp.float32)
         + jnp.dot(agg_lo, wnlo_ref[...], preferred_element_type=jnp.float32)
         + jnp.dot(agg_hi, wnhi_ref[...], preferred_element_type=jnp.float32)
         + b_ref[...])
    return jnp.maximum(h, 0.0)


def _tc0_body(x_ref, plo_ref, phi_ref, deg_ref, ws_ref, wnlo_ref, wnhi_ref,
              b_ref, h_ref, hlo_ref, hhi_ref):
    h = _hidden_sage(x_ref, plo_ref, phi_ref, deg_ref, ws_ref, wnlo_ref,
                     wnhi_ref, b_ref)
    h_ref[...] = h
    hlo_ref[...] = h[:, :_DH]
    hhi_ref[...] = h[:, _DH:]


def _tc1_body(x_ref, plo_ref, phi_ref, deg_ref, ws_ref, wnlo_ref, wnhi_ref,
              b_ref, wn2_ref, h_ref, y2_ref):
    h = _hidden_sage(x_ref, plo_ref, phi_ref, deg_ref, ws_ref, wnlo_ref,
                     wnhi_ref, b_ref)
    h_ref[...] = h
    y2_ref[...] = jnp.dot(h, wn2_ref[...], preferred_element_type=jnp.float32)


def _tc2_body(h_ref, p_ref, deg_ref, ws_ref, b_ref, out_ref):
    rdeg = 1.0 / jnp.maximum(deg_ref[...], 1.0)
    out_ref[...] = (
        jnp.dot(h_ref[...], ws_ref[...], preferred_element_type=jnp.float32)
        + (p_ref[0] + p_ref[1]) * rdeg + b_ref[...])


_HIDDEN_SPECS = [
    pl.BlockSpec((_BLK, IN_FEATS), lambda i: (i, 0)),
    pl.BlockSpec((2, _BLK, _DH), lambda i: (0, i, 0)),
    pl.BlockSpec((2, _BLK, _DH), lambda i: (0, i, 0)),
    pl.BlockSpec((_BLK, 1), lambda i: (i, 0)),
    pl.BlockSpec((IN_FEATS, N_HIDDEN), lambda i: (0, 0)),
    pl.BlockSpec((_DH, N_HIDDEN), lambda i: (0, 0)),
    pl.BlockSpec((_DH, N_HIDDEN), lambda i: (0, 0)),
    pl.BlockSpec((1, N_HIDDEN), lambda i: (0, 0)),
]

_tc0 = pl.pallas_call(
    _tc0_body,
    grid=_GRID,
    in_specs=_HIDDEN_SPECS,
    out_specs=[
        pl.BlockSpec((_BLK, N_HIDDEN), lambda i: (i, 0)),
        pl.BlockSpec((_BLK, _DH), lambda i: (i, 0)),
        pl.BlockSpec((_BLK, _DH), lambda i: (i, 0)),
    ],
    out_shape=[
        jax.ShapeDtypeStruct((_NPAD, N_HIDDEN), jnp.float32),
        jax.ShapeDtypeStruct((_NPAD, _DH), jnp.float32),
        jax.ShapeDtypeStruct((_NPAD, _DH), jnp.float32),
    ],
)

_tc1 = pl.pallas_call(
    _tc1_body,
    grid=_GRID,
    in_specs=_HIDDEN_SPECS + [
        pl.BlockSpec((N_HIDDEN, N_CLASSES), lambda i: (0, 0)),
    ],
    out_specs=[
        pl.BlockSpec((_BLK, N_HIDDEN), lambda i: (i, 0)),
        pl.BlockSpec((_BLK, N_CLASSES), lambda i: (i, 0)),
    ],
    out_shape=[
        jax.ShapeDtypeStruct((_NPAD, N_HIDDEN), jnp.float32),
        jax.ShapeDtypeStruct((_NPAD, N_CLASSES), jnp.float32),
    ],
)

_tc2 = pl.pallas_call(
    _tc2_body,
    grid=_GRID,
    in_specs=[
        pl.BlockSpec((_BLK, N_HIDDEN), lambda i: (i, 0)),
        pl.BlockSpec((2, _BLK, N_CLASSES), lambda i: (0, i, 0)),
        pl.BlockSpec((_BLK, 1), lambda i: (i, 0)),
        pl.BlockSpec((N_HIDDEN, N_CLASSES), lambda i: (0, 0)),
        pl.BlockSpec((1, N_CLASSES), lambda i: (0, 0)),
    ],
    out_specs=pl.BlockSpec((_BLK, N_CLASSES), lambda i: (i, 0)),
    out_shape=jax.ShapeDtypeStruct((_NPAD, N_CLASSES), jnp.float32),
)


def kernel(x, edge_index, Ws0, Wn0, b0, Ws1, Wn1, b1, Ws2, Wn2, b2):
    npad = _EPAD - N_EDGES
    src = jnp.concatenate(
        [edge_index[0].astype(jnp.int32), jnp.zeros((npad,), jnp.int32)])
    dst = jnp.concatenate(
        [edge_index[1].astype(jnp.int32),
         jnp.full((npad,), _NPAD - 1, jnp.int32)])
    zeros = jnp.zeros((_RPT, _DH), jnp.float32)
    x_pad = jnp.zeros((_NPAD, IN_FEATS), jnp.float32).at[:N_NODES].set(x)

    p0lo, p0hi, deg_parts = _agg_deg(
        x_pad[:, :_DH], x_pad[:, _DH:], src, dst, zeros)
    deg_row = _tc_deg(deg_parts.reshape(_NW, _NPAD))
    deg_col = deg_row.reshape(_NPAD, 1)

    h0, h0lo, h0hi = _tc0(x_pad, p0lo, p0hi, deg_col, Ws0,
                          Wn0[:_DH], Wn0[_DH:], b0.reshape(1, -1))
    p1lo, p1hi = _agg(h0lo, h0hi, src, dst, zeros)
    h1, y2 = _tc1(h0, p1lo, p1hi, deg_col, Ws1,
                  Wn1[:_DH], Wn1[_DH:], b1.reshape(1, -1), Wn2)
    p2, = _agg1(y2, src, dst, zeros)
    out = _tc2(h1, p2, deg_col, Ws2, b2.reshape(1, -1))
    return out[:N_NODES]


# speedup vs baseline: 1.0976x; 1.0976x over previous
import functools

import jax
import jax.numpy as jnp
from jax import lax
from jax.experimental import pallas as pl
from jax.experimental.pallas import tpu as pltpu
from jax.experimental.pallas import tpu_sc as plsc

N_NODES = 10000
N_EDGES = 320000
IN_FEATS = 128
N_HIDDEN = 128
N_CLASSES = 64

_NC = 2
_NS = 16
_NW = _NC * _NS
_CH = 128
_NCH = 80
_EPT = _NCH * _CH
_EPAD = _NW * _EPT
_NPAD = 10240
_RPT = _NPAD // _NS
_DH = 64


def _make_agg(with_deg, n_halves=2):
    mesh = plsc.VectorSubcoreMesh(core_axis_name="c", subcore_axis_name="s")

    out_type = [jax.ShapeDtypeStruct((_NC, _NPAD, _DH), jnp.float32)
                for _ in range(n_halves)]
    scratch = [
        pltpu.VMEM((_EPT,), jnp.int32),
        pltpu.VMEM((_EPT,), jnp.int32),
        pltpu.VMEM((_CH,), jnp.int32),
        pltpu.VMEM((_CH, _DH), jnp.float32),
        pltpu.VMEM((_CH, _DH), jnp.float32),
        pltpu.VMEM_SHARED((_NPAD, _DH), jnp.float32),
        pltpu.VMEM_SHARED((_NPAD, _DH), jnp.float32),
        pltpu.SemaphoreType.DMA,
        pltpu.SemaphoreType.DMA,
    ]
    if with_deg:
        out_type.append(jax.ShapeDtypeStruct((_NW * _NPAD,), jnp.float32))
        scratch.append(pltpu.VMEM((_NPAD,), jnp.float32))

    @functools.partial(
        pl.kernel,
        out_type=out_type,
        mesh=mesh,
        scratch_types=scratch,
        compiler_params=pltpu.CompilerParams(
            needs_layout_passes=False, use_tc_tiling_on_sc=False),
    )
    def agg(*args):
        t_hbms = args[:n_halves]
        src_hbm, dst_hbm, zeros_hbm = args[n_halves:n_halves + 3]
        refs = args[n_halves + 3:]
        o_hbms = refs[:n_halves]
        refs = refs[n_halves:]
        if with_deg:
            deg_hbm, srcf, dstf, dstbuf, rows0, rows1, \
                table, accum, sem0, sem1, degv = refs
        else:
            srcf, dstf, dstbuf, rows0, rows1, \
                table, accum, sem0, sem1 = refs
        c = lax.axis_index("c")
        s = lax.axis_index("s")
        wid = s * _NC + c
        myrows = pl.ds(s * _RPT, _RPT)
        ones16 = jnp.ones((16,), jnp.float32)

        def gather(i, buf, sem):
            idx = srcf.at[pl.ds(pl.multiple_of(i * _CH, _CH), _CH)]
            pltpu.async_copy(table.at[idx], buf, sem)

        def gwait(i, buf, sem):
            idx = srcf.at[pl.ds(pl.multiple_of(i * _CH, _CH), _CH)]
            pltpu.make_async_copy(table.at[idx], buf, sem).wait()

        for half in range(n_halves):
            t_hbm = t_hbms[half]
            out_hbm = o_hbms[half]
            pltpu.sync_copy(t_hbm.at[myrows], table.at[myrows])
            pltpu.sync_copy(zeros_hbm, accum.at[myrows])
            if half == 0:
                pltpu.sync_copy(src_hbm.at[pl.ds(wid * _EPT, _EPT)], srcf)
                pltpu.sync_copy(dst_hbm.at[pl.ds(wid * _EPT, _EPT)], dstf)
                if with_deg:
                    def zstep(i, carry):
                        degv[pl.ds(i * 16, 16)] = jnp.zeros((16,), jnp.float32)
                        return carry
                    lax.fori_loop(0, _NPAD // 16, zstep, 0)
            plsc.subcore_barrier()

            def consume(i, buf, track_deg):
                for j in range(_CH // 16):
                    v = dstf[pl.ds(i * _CH + j * 16, 16)]
                    dstbuf[pl.ds(j * 16, 16)] = v
                    if track_deg:
                        plsc.addupdate_scatter(degv, [v], ones16)
                pltpu.sync_copy(buf, accum.at[dstbuf], add=True)

            track_deg = with_deg and half == 0
            gather(0, rows0, sem0)

            def pair(p, carry):
                i0 = 2 * p
                i1 = i0 + 1
                gather(i1, rows1, sem1)
                gwait(i0, rows0, sem0)
                consume(i0, rows0, track_deg)

                @pl.when(p < _NCH // 2 - 1)
                def _():
                    gather(i0 + 2, rows0, sem0)

                gwait(i1, rows1, sem1)
                consume(i1, rows1, track_deg)
                return carry

            lax.fori_loop(0, _NCH // 2, pair, 0)
            plsc.subcore_barrier()
            pltpu.sync_copy(accum.at[myrows], out_hbm.at[c, myrows])

        if with_deg:
            pltpu.sync_copy(degv, deg_hbm.at[pl.ds(wid * _NPAD, _NPAD)])

    return agg


_agg_deg = _make_agg(True)
_agg = _make_agg(False)
_agg1 = _make_agg(False, n_halves=1)

_BLK = 1024
_GRID = (_NPAD // _BLK,)


def _deg_body(parts_ref, deg_ref):
    deg_ref[...] = jnp.sum(parts_ref[...], axis=0, keepdims=True)


_tc_deg = pl.pallas_call(
    _deg_body,
    grid=_GRID,
    in_specs=[pl.BlockSpec((_NW, _BLK), lambda i: (0, i))],
    out_specs=pl.BlockSpec((1, _BLK), lambda i: (0, i)),
    out_shape=jax.ShapeDtypeStruct((1, _NPAD), jnp.float32),
)


def _hidden_sage(x_ref, plo_ref, phi_ref, deg_ref, ws_ref, wnlo_ref,
                 wnhi_ref, b_ref):
    rdeg = 1.0 / jnp.maximum(deg_ref[...], 1.0)
    agg_lo = (plo_ref[0] + plo_ref[1]) * rdeg
    agg_hi = (phi_ref[0] + phi_ref[1]) * rdeg
    h = (jnp.dot(x_ref[...], ws_ref[...], preferred_element_type=jnp.float32)
         + jnp.dot(agg_lo, wnlo_ref[...], preferred_element_type=jnp.float32)
         + jnp.dot(agg_hi, wnhi_ref[...], preferred_element_type=jnp.float32)
         + b_ref[...])
    return jnp.maximum(h, 0.0)


def _tc0_body(x_ref, plo_ref, phi_ref, deg_ref, ws_ref, wnlo_ref, wnhi_ref,
              b_ref, h_ref, hlo_ref, hhi_ref):
    h = _hidden_sage(x_ref, plo_ref, phi_ref, deg_ref, ws_ref, wnlo_ref,
                     wnhi_ref, b_ref)
    h_ref[...] = h
    hlo_ref[...] = h[:, :_DH]
    hhi_ref[...] = h[:, _DH:]


def _tc1_body(x_ref, plo_ref, phi_ref, deg_ref, ws_ref, wnlo_ref, wnhi_ref,
              b_ref, wn2_ref, h_ref, y2_ref):
    h = _hidden_sage(x_ref, plo_ref, phi_ref, deg_ref, ws_ref, wnlo_ref,
                     wnhi_ref, b_ref)
    h_ref[...] = h
    y2_ref[...] = jnp.dot(h, wn2_ref[...], preferred_element_type=jnp.float32)


def _tc2_body(h_ref, p_ref, deg_ref, ws_ref, b_ref, out_ref):
    rdeg = 1.0 / jnp.maximum(deg_ref[...], 1.0)
    out_ref[...] = (
        jnp.dot(h_ref[...], ws_ref[...], preferred_element_type=jnp.float32)
        + (p_ref[0] + p_ref[1]) * rdeg + b_ref[...])


_HIDDEN_SPECS = [
    pl.BlockSpec((_BLK, IN_FEATS), lambda i: (i, 0)),
    pl.BlockSpec((2, _BLK, _DH), lambda i: (0, i, 0)),
    pl.BlockSpec((2, _BLK, _DH), lambda i: (0, i, 0)),
    pl.BlockSpec((_BLK, 1), lambda i: (i, 0)),
    pl.BlockSpec((IN_FEATS, N_HIDDEN), lambda i: (0, 0)),
    pl.BlockSpec((_DH, N_HIDDEN), lambda i: (0, 0)),
    pl.BlockSpec((_DH, N_HIDDEN), lambda i: (0, 0)),
    pl.BlockSpec((1, N_HIDDEN), lambda i: (0, 0)),
]

_tc0 = pl.pallas_call(
    _tc0_body,
    grid=_GRID,
    in_specs=_HIDDEN_SPECS,
    out_specs=[
        pl.BlockSpec((_BLK, N_HIDDEN), lambda i: (i, 0)),
        pl.BlockSpec((_BLK, _DH), lambda i: (i, 0)),
        pl.BlockSpec((_BLK, _DH), lambda i: (i, 0)),
    ],
    out_shape=[
        jax.ShapeDtypeStruct((_NPAD, N_HIDDEN), jnp.float32),
        jax.ShapeDtypeStruct((_NPAD, _DH), jnp.float32),
        jax.ShapeDtypeStruct((_NPAD, _DH), jnp.float32),
    ],
)

_tc1 = pl.pallas_call(
    _tc1_body,
    grid=_GRID,
    in_specs=_HIDDEN_SPECS + [
        pl.BlockSpec((N_HIDDEN, N_CLASSES), lambda i: (0, 0)),
    ],
    out_specs=[
        pl.BlockSpec((_BLK, N_HIDDEN), lambda i: (i, 0)),
        pl.BlockSpec((_BLK, N_CLASSES), lambda i: (i, 0)),
    ],
    out_shape=[
        jax.ShapeDtypeStruct((_NPAD, N_HIDDEN), jnp.float32),
        jax.ShapeDtypeStruct((_NPAD, N_CLASSES), jnp.float32),
    ],
)

_tc2 = pl.pallas_call(
    _tc2_body,
    grid=_GRID,
    in_specs=[
        pl.BlockSpec((_BLK, N_HIDDEN), lambda i: (i, 0)),
        pl.BlockSpec((2, _BLK, N_CLASSES), lambda i: (0, i, 0)),
        pl.BlockSpec((_BLK, 1), lambda i: (i, 0)),
        pl.BlockSpec((N_HIDDEN, N_CLASSES), lambda i: (0, 0)),
        pl.BlockSpec((1, N_CLASSES), lambda i: (0, 0)),
    ],
    out_specs=pl.BlockSpec((_BLK, N_CLASSES), lambda i: (i, 0)),
    out_shape=jax.ShapeDtypeStruct((_NPAD, N_CLASSES), jnp.float32),
)


def kernel(x, edge_index, Ws0, Wn0, b0, Ws1, Wn1, b1, Ws2, Wn2, b2):
    npad = _EPAD - N_EDGES
    src = jnp.concatenate(
        [edge_index[0].astype(jnp.int32), jnp.zeros((npad,), jnp.int32)])
    dst = jnp.concatenate(
        [edge_index[1].astype(jnp.int32),
         jnp.full((npad,), _NPAD - 1, jnp.int32)])
    zeros = jnp.zeros((_RPT, _DH), jnp.float32)
    x_pad = jnp.zeros((_NPAD, IN_FEATS), jnp.float32).at[:N_NODES].set(x)

    p0lo, p0hi, deg_parts = _agg_deg(
        x_pad[:, :_DH], x_pad[:, _DH:], src, dst, zeros)
    deg_row = _tc_deg(deg_parts.reshape(_NW, _NPAD))
    deg_col = deg_row.reshape(_NPAD, 1)

    h0, h0lo, h0hi = _tc0(x_pad, p0lo, p0hi, deg_col, Ws0,
                          Wn0[:_DH], Wn0[_DH:], b0.reshape(1, -1))
    p1lo, p1hi = _agg(h0lo, h0hi, src, dst, zeros)
    h1, y2 = _tc1(h0, p1lo, p1hi, deg_col, Ws1,
                  Wn1[:_DH], Wn1[_DH:], b1.reshape(1, -1), Wn2)
    p2, = _agg1(y2, src, dst, zeros)
    out = _tc2(h1, p2, deg_col, Ws2, b2.reshape(1, -1))
    return out[:N_NODES]
